# Initial kernel scaffold; baseline (speedup 1.0000x reference)
import jax
import jax.numpy as jnp
from jax.experimental import pallas as pl


def kernel(token_ids, token_embedding):
    raise NotImplementedError("write your pallas kernel here")


# speedup vs baseline: 4.5753x; 4.5753x over previous
import functools
import math

import jax
import jax.numpy as jnp
from jax import lax
from jax.experimental import pallas as pl
from jax.experimental.pallas import tpu as pltpu
from jax.experimental.pallas import tpu_sc as plsc

NUM_CORES = 2
NUM_SUBCORES = 16
NW = NUM_CORES * NUM_SUBCORES

MAX_CTX_LEN = 256


def _positional_encoding(seq_len: int, d: int) -> jnp.ndarray:
    full_len = max(seq_len, MAX_CTX_LEN)
    position = jnp.arange(full_len, dtype=jnp.float32)[:, None]
    div_term = jnp.exp(
        jnp.arange(0, d, 2, dtype=jnp.float32) * (-math.log(10000.0) / d)
    )
    pe = jnp.zeros((full_len, d), dtype=jnp.float32)
    pe = pe.at[:, 0::2].set(jnp.sin(position * div_term))
    pe = pe.at[:, 1::2].set(jnp.cos(position * div_term))
    return pe[:seq_len]


@functools.lru_cache(maxsize=None)
def _make_kernel(B: int, S: int, D: int):
    rows_per_w = B // NW
    mesh = plsc.VectorSubcoreMesh(
        core_axis_name="c",
        subcore_axis_name="s",
        num_cores=NUM_CORES,
        num_subcores=NUM_SUBCORES,
    )

    @functools.partial(
        pl.kernel,
        out_type=jax.ShapeDtypeStruct((B, S, D), jnp.float32),
        mesh=mesh,
        scratch_types=[
            pltpu.VMEM((S,), jnp.int32),
            pltpu.VMEM_SHARED((S, D), jnp.float32),
            pltpu.VMEM((S, D), jnp.float32),
            pltpu.SemaphoreType.DMA,
        ],
    )
    def emb_kernel(ids_hbm, table_hbm, pe_hbm, out_hbm, idx_v, pe_sh, rows_v, sem):
        sid = lax.axis_index("s")
        wid = sid * NUM_CORES + lax.axis_index("c")

        @pl.when(sid == 0)
        def _load_pe():
            pltpu.sync_copy(pe_hbm, pe_sh)

        plsc.subcore_barrier()

        def body(i, carry):
            r = wid * rows_per_w + i
            pltpu.sync_copy(ids_hbm.at[r], idx_v)
            pltpu.sync_copy(pe_sh, rows_v)
            pltpu.async_copy(table_hbm.at[idx_v], rows_v, sem, add=True).wait()
            pltpu.sync_copy(rows_v, out_hbm.at[r])
            return carry

        lax.fori_loop(0, rows_per_w, body, 0)

    return emb_kernel


def kernel(token_ids, token_embedding):
    B, S = token_ids.shape
    V, D = token_embedding.shape
    pe = _positional_encoding(S, D)
    ids = token_ids.astype(jnp.int32)
    emb_kernel = _make_kernel(B, S, D)
    return emb_kernel(ids, token_embedding, pe)


# speedup vs baseline: 7.3033x; 1.5962x over previous
import functools
import math

import jax
import jax.numpy as jnp
from jax import lax
from jax.experimental import pallas as pl
from jax.experimental.pallas import tpu as pltpu
from jax.experimental.pallas import tpu_sc as plsc

NUM_CORES = 2
NUM_SUBCORES = 16
NW = NUM_CORES * NUM_SUBCORES

MAX_CTX_LEN = 256

CH_ROWS = 2
NBUF = 2


def _positional_encoding(seq_len: int, d: int) -> jnp.ndarray:
    full_len = max(seq_len, MAX_CTX_LEN)
    position = jnp.arange(full_len, dtype=jnp.float32)[:, None]
    div_term = jnp.exp(
        jnp.arange(0, d, 2, dtype=jnp.float32) * (-math.log(10000.0) / d)
    )
    pe = jnp.zeros((full_len, d), dtype=jnp.float32)
    pe = pe.at[:, 0::2].set(jnp.sin(position * div_term))
    pe = pe.at[:, 1::2].set(jnp.cos(position * div_term))
    return pe[:seq_len]


@functools.lru_cache(maxsize=None)
def _make_kernel(B: int, S: int, D: int):
    per_w = B * S // NW
    ch = CH_ROWS * S
    nsteps = per_w // ch
    assert per_w % ch == 0 and nsteps % NBUF == 0
    mesh = plsc.VectorSubcoreMesh(
        core_axis_name="c",
        subcore_axis_name="s",
        num_cores=NUM_CORES,
        num_subcores=NUM_SUBCORES,
    )

    @functools.partial(
        pl.kernel,
        out_type=jax.ShapeDtypeStruct((B * S, D), jnp.float32),
        mesh=mesh,
        scratch_types=[
            pltpu.VMEM((per_w,), jnp.int32),
            pltpu.VMEM_SHARED((ch, D), jnp.float32),
            pltpu.VMEM((NBUF, ch, D), jnp.float32),
            pltpu.SemaphoreType.DMA,
        ]
        + [pltpu.SemaphoreType.DMA] * NBUF,
    )
    def emb_kernel(ids_hbm, table_hbm, pe_hbm, out_hbm, *scratch):
        idx_all = scratch[0]
        pe_sh = scratch[1]
        bufs = scratch[2]
        sem_g = scratch[3]
        sem_wb = scratch[4:]
        sid = lax.axis_index("s")
        wid = sid * NUM_CORES + lax.axis_index("c")
        base = wid * per_w

        @pl.when(sid == 0)
        def _load_pe():
            pltpu.sync_copy(pe_hbm, pe_sh)

        pltpu.sync_copy(ids_hbm.at[pl.ds(base, per_w)], idx_all)
        plsc.subcore_barrier()

        def prefill(b):
            pltpu.sync_copy(pe_sh, bufs.at[b])

        def out_slice(k):
            return out_hbm.at[pl.ds(base + k * ch, ch)]

        def wait_wb(k, b):
            pltpu.make_async_copy(bufs.at[b], out_slice(k), sem_wb[b]).wait()

        prefill(0)

        def outer(g, carry):
            for b in range(NBUF):
                k = g * NBUF + b
                nb = (b + 1) % NBUF
                gather = pltpu.async_copy(
                    table_hbm.at[idx_all.at[pl.ds(k * ch, ch)]],
                    bufs.at[b],
                    sem_g,
                    add=True,
                )

                @pl.when(k + 1 < nsteps)
                def _stage():
                    @pl.when(k >= 1)
                    def _recycle():
                        wait_wb(k - 1, nb)

                    prefill(nb)

                gather.wait()
                pltpu.async_copy(bufs.at[b], out_slice(k), sem_wb[b])

            return carry

        lax.fori_loop(0, nsteps // NBUF, outer, 0)

        wait_wb(nsteps - 2, (nsteps - 2) % NBUF)
        wait_wb(nsteps - 1, (nsteps - 1) % NBUF)

    return emb_kernel


def kernel(token_ids, token_embedding):
    B, S = token_ids.shape
    V, D = token_embedding.shape
    pe = _positional_encoding(S, D)
    pe_rep = jnp.tile(pe, (CH_ROWS, 1))
    ids = token_ids.reshape(B * S).astype(jnp.int32)
    emb_kernel = _make_kernel(B, S, D)
    out = emb_kernel(ids, token_embedding, pe_rep)
    return out.reshape(B, S, D)


# speedup vs baseline: 7.6809x; 1.0517x over previous
import functools
import math

import jax
import jax.numpy as jnp
from jax import lax
from jax.experimental import pallas as pl
from jax.experimental.pallas import tpu as pltpu
from jax.experimental.pallas import tpu_sc as plsc

NUM_CORES = 2
NUM_SUBCORES = 16
NW = NUM_CORES * NUM_SUBCORES

MAX_CTX_LEN = 256

CH_ROWS = 1
NBUF = 4


def _positional_encoding(seq_len: int, d: int) -> jnp.ndarray:
    full_len = max(seq_len, MAX_CTX_LEN)
    position = jnp.arange(full_len, dtype=jnp.float32)[:, None]
    div_term = jnp.exp(
        jnp.arange(0, d, 2, dtype=jnp.float32) * (-math.log(10000.0) / d)
    )
    pe = jnp.zeros((full_len, d), dtype=jnp.float32)
    pe = pe.at[:, 0::2].set(jnp.sin(position * div_term))
    pe = pe.at[:, 1::2].set(jnp.cos(position * div_term))
    return pe[:seq_len]


@functools.lru_cache(maxsize=None)
def _make_kernel(B: int, S: int, D: int):
    per_w = B * S // NW
    ch = CH_ROWS * S
    nsteps = per_w // ch
    assert per_w % ch == 0 and nsteps % NBUF == 0
    mesh = plsc.VectorSubcoreMesh(
        core_axis_name="c",
        subcore_axis_name="s",
        num_cores=NUM_CORES,
        num_subcores=NUM_SUBCORES,
    )

    @functools.partial(
        pl.kernel,
        out_type=jax.ShapeDtypeStruct((B * S, D), jnp.float32),
        mesh=mesh,
        scratch_types=[
            pltpu.VMEM((per_w,), jnp.int32),
            pltpu.VMEM_SHARED((ch, D), jnp.float32),
            pltpu.VMEM((NBUF, ch, D), jnp.float32),
        ]
        + [pltpu.SemaphoreType.DMA] * (2 * NBUF),
    )
    def emb_kernel(ids_hbm, table_hbm, pe_hbm, out_hbm, *scratch):
        idx_all = scratch[0]
        pe_sh = scratch[1]
        bufs = scratch[2]
        sem_g = scratch[3 : 3 + NBUF]
        sem_wb = scratch[3 + NBUF :]
        sid = lax.axis_index("s")
        wid = sid * NUM_CORES + lax.axis_index("c")
        base = wid * per_w

        @pl.when(sid == 0)
        def _load_pe():
            pltpu.sync_copy(pe_hbm, pe_sh)

        pltpu.sync_copy(ids_hbm.at[pl.ds(base, per_w)], idx_all)
        plsc.subcore_barrier()

        def prefill(b):
            pltpu.sync_copy(pe_sh, bufs.at[b])

        def out_slice(k):
            return out_hbm.at[pl.ds(base + k * ch, ch)]

        def issue_gather(k, b):
            pltpu.async_copy(
                table_hbm.at[idx_all.at[pl.ds(k * ch, ch)]],
                bufs.at[b],
                sem_g[b],
                add=True,
            )

        def wait_gather(k, b):
            pltpu.make_async_copy(
                table_hbm.at[idx_all.at[pl.ds(k * ch, ch)]], bufs.at[b], sem_g[b]
            ).wait()

        def wait_wb(k, b):
            pltpu.make_async_copy(bufs.at[b], out_slice(k), sem_wb[b]).wait()

        prefill(0)
        issue_gather(0, 0)
        prefill(1)
        issue_gather(1, 1)

        def outer(g, carry):
            for b in range(NBUF):
                k = g * NBUF + b
                b2 = (b + 2) % NBUF
                wait_gather(k, b)
                pltpu.async_copy(bufs.at[b], out_slice(k), sem_wb[b])

                @pl.when(k + 2 < nsteps)
                def _stage():
                    @pl.when(k >= 2)
                    def _recycle():
                        wait_wb(k - 2, b2)

                    prefill(b2)
                    issue_gather(k + 2, b2)

            return carry

        lax.fori_loop(0, nsteps // NBUF, outer, 0)

        wait_wb(nsteps - 2, (nsteps - 2) % NBUF)
        wait_wb(nsteps - 1, (nsteps - 1) % NBUF)

    return emb_kernel


def kernel(token_ids, token_embedding):
    B, S = token_ids.shape
    V, D = token_embedding.shape
    pe = _positional_encoding(S, D)
    pe_rep = jnp.tile(pe, (CH_ROWS, 1))
    ids = token_ids.reshape(B * S).astype(jnp.int32)
    emb_kernel = _make_kernel(B, S, D)
    out = emb_kernel(ids, token_embedding, pe_rep)
    return out.reshape(B, S, D)
